# Initial kernel scaffold; baseline (speedup 1.0000x reference)
import jax
import jax.numpy as jnp
from jax.experimental import pallas as pl


def kernel(xv, adj_pos, adj_neg):
    raise NotImplementedError("write your pallas kernel here")


# speedup vs baseline: 83.1650x; 83.1650x over previous
import functools

import jax
import jax.numpy as jnp
from jax import lax
from jax.experimental import pallas as pl
from jax.experimental.pallas import tpu as pltpu
from jax.experimental.pallas import tpu_sc as plsc

N_VARS = 100000
N_CLAUSES = 100000
E = 3200000

NC = 2
NS = 16
L = 16
NW = NC * NS

VPAD = 102400
WORDS = VPAD // 32
CPAD = 102400
EPW = E // NW
CHUNK = 4000
NCHUNK = EPW // CHUNK
VECS = CHUNK // L
THRESH = 0.50001
RBLK = CPAD // 8


def _pack_body(x_ref, out_ref):
    x = x_ref[...]
    b = jnp.where(x >= THRESH, 1, 0).astype(jnp.int32)
    shifts = lax.broadcasted_iota(jnp.int32, (32, 1), 0)
    out_ref[...] = jnp.sum(b << shifts, axis=0, keepdims=True)


_pack_call = pl.pallas_call(
    _pack_body,
    out_shape=jax.ShapeDtypeStruct((1, WORDS), jnp.int32),
)


_sc_mesh = plsc.VectorSubcoreMesh(core_axis_name="c", subcore_axis_name="s")


@functools.partial(
    pl.kernel,
    out_type=jax.ShapeDtypeStruct((NW, CPAD), jnp.int32),
    mesh=_sc_mesh,
    compiler_params=pltpu.CompilerParams(needs_layout_passes=False),
    scratch_types=[
        pltpu.VMEM((WORDS,), jnp.int32),
        pltpu.VMEM((CPAD,), jnp.int32),
        pltpu.VMEM((CHUNK,), jnp.int32),
        pltpu.VMEM((CHUNK,), jnp.int32),
    ],
)
def _scatter_kernel(packed_hbm, pos_hbm, neg_hbm, out_hbm,
                    packed_v, acc_v, ic_v, iv_v):
    wid = lax.axis_index("s") * NC + lax.axis_index("c")
    base = wid * EPW
    pltpu.sync_copy(packed_hbm, packed_v)

    zeros = jnp.zeros((L,), jnp.int32)

    def zbody(i, _):
        acc_v[pl.ds(i * L, L)] = zeros
        return 0

    lax.fori_loop(0, CPAD // L, zbody, 0)

    for adj_hbm, is_pos in ((pos_hbm, True), (neg_hbm, False)):
        def cbody(k, _, adj_hbm=adj_hbm, is_pos=is_pos):
            off = base + k * CHUNK
            pltpu.sync_copy(adj_hbm.at[pl.ds(off, CHUNK)], ic_v)
            pltpu.sync_copy(adj_hbm.at[pl.ds(E + off, CHUNK)], iv_v)

            def vbody(i, _):
                iv = iv_v[pl.ds(i * L, L)]
                ic = ic_v[pl.ds(i * L, L)]
                word = plsc.load_gather(packed_v, [lax.rem(iv, WORDS)])
                sh = lax.div(iv, WORDS)
                b = lax.shift_right_logical(word, sh) & 1
                val = b if is_pos else 1 - b
                plsc.addupdate_scatter(acc_v, [ic], val)
                return 0

            lax.fori_loop(0, VECS, vbody, 0)
            return 0

        lax.fori_loop(0, NCHUNK, cbody, 0)

    pltpu.sync_copy(acc_v, out_hbm.at[wid])


def _reduce_body(x_ref, out_ref):
    j = pl.program_id(0)
    s = jnp.sum(x_ref[...], axis=0, keepdims=True)
    cid = j * RBLK + lax.broadcasted_iota(jnp.int32, (1, RBLK), 1)
    s = jnp.where(cid < N_CLAUSES, s, jnp.int32(2**31 - 1))
    m = jnp.min(s)

    @pl.when(j == 0)
    def _():
        out_ref[0, 0] = m

    @pl.when(j > 0)
    def _():
        out_ref[0, 0] = jnp.minimum(out_ref[0, 0], m)


_reduce_call = pl.pallas_call(
    _reduce_body,
    grid=(CPAD // RBLK,),
    in_specs=[pl.BlockSpec((NW, RBLK), lambda j: (0, j))],
    out_specs=pl.BlockSpec(memory_space=pltpu.SMEM),
    out_shape=jax.ShapeDtypeStruct((1, 1), jnp.int32),
)


def kernel(xv, adj_pos, adj_neg):
    xvp = jnp.pad(xv, (0, VPAD - N_VARS)).reshape(32, WORDS)
    packed = _pack_call(xvp).reshape(WORDS)
    partials = _scatter_kernel(packed, adj_pos.reshape(2 * E),
                               adj_neg.reshape(2 * E))
    m = _reduce_call(partials)
    return m[0, 0].astype(jnp.float32)


# speedup vs baseline: 200.9677x; 2.4165x over previous
import functools

import jax
import jax.numpy as jnp
from jax import lax
from jax.experimental import pallas as pl
from jax.experimental.pallas import tpu as pltpu
from jax.experimental.pallas import tpu_sc as plsc

N_VARS = 100000
N_CLAUSES = 100000
E = 3200000

NC = 2
NS = 16
L = 16
NW = NC * NS

WORDS = 4096
LOG2W = WORDS.bit_length() - 1
VPAD = 32 * WORDS
CPAD = 102400
EPW = E // NW
CHUNK = 4000
NCHUNK = EPW // CHUNK
UNROLL = 10
VECS = CHUNK // (L * UNROLL)
THRESH = 0.50001
RBLK = CPAD // 8


def _pack_body(x_ref, out_ref):
    x = x_ref[...]
    b = jnp.where(x >= THRESH, 1, 0).astype(jnp.int32)
    shifts = lax.broadcasted_iota(jnp.int32, (32, 1), 0)
    out_ref[...] = jnp.sum(b << shifts, axis=0, keepdims=True)


_pack_call = pl.pallas_call(
    _pack_body,
    out_shape=jax.ShapeDtypeStruct((1, WORDS), jnp.int32),
)


_sc_mesh = plsc.VectorSubcoreMesh(core_axis_name="c", subcore_axis_name="s")


@functools.partial(
    pl.kernel,
    out_type=jax.ShapeDtypeStruct((NW, CPAD), jnp.int32),
    mesh=_sc_mesh,
    compiler_params=pltpu.CompilerParams(needs_layout_passes=False),
    scratch_types=[
        pltpu.VMEM((WORDS,), jnp.int32),
        pltpu.VMEM((CPAD,), jnp.int32),
        pltpu.VMEM((CHUNK,), jnp.int32),
        pltpu.VMEM((CHUNK,), jnp.int32),
    ],
)
def _scatter_kernel(packed_hbm, pos_hbm, neg_hbm, out_hbm,
                    packed_v, acc_v, ic_v, iv_v):
    wid = lax.axis_index("s") * NC + lax.axis_index("c")
    base = wid * EPW
    pltpu.sync_copy(packed_hbm, packed_v)

    zeros = jnp.zeros((L,), jnp.int32)

    def zbody(i, _):
        acc_v[pl.ds(i * L, L)] = zeros
        return 0

    lax.fori_loop(0, CPAD // L, zbody, 0)

    for adj_hbm, is_pos in ((pos_hbm, True), (neg_hbm, False)):
        def cbody(k, _, adj_hbm=adj_hbm, is_pos=is_pos):
            off = base + k * CHUNK
            pltpu.sync_copy(adj_hbm.at[pl.ds(off, CHUNK)], ic_v)
            pltpu.sync_copy(adj_hbm.at[pl.ds(E + off, CHUNK)], iv_v)

            def vbody(i, _):
                for u in range(UNROLL):
                    o = i * (L * UNROLL) + u * L
                    iv = iv_v[pl.ds(o, L)]
                    ic = ic_v[pl.ds(o, L)]
                    word = plsc.load_gather(packed_v, [iv & (WORDS - 1)])
                    sh = lax.shift_right_logical(iv, LOG2W)
                    b = lax.shift_right_logical(word, sh) & 1
                    val = b if is_pos else 1 - b
                    plsc.addupdate_scatter(acc_v, [ic], val)
                return 0

            lax.fori_loop(0, VECS, vbody, 0)
            return 0

        lax.fori_loop(0, NCHUNK, cbody, 0)

    pltpu.sync_copy(acc_v, out_hbm.at[wid])


def _reduce_body(x_ref, out_ref):
    j = pl.program_id(0)
    s = jnp.sum(x_ref[...], axis=0, keepdims=True)
    cid = j * RBLK + lax.broadcasted_iota(jnp.int32, (1, RBLK), 1)
    s = jnp.where(cid < N_CLAUSES, s, jnp.int32(2**31 - 1))
    m = jnp.min(s)

    @pl.when(j == 0)
    def _():
        out_ref[0, 0] = m

    @pl.when(j > 0)
    def _():
        out_ref[0, 0] = jnp.minimum(out_ref[0, 0], m)


_reduce_call = pl.pallas_call(
    _reduce_body,
    grid=(CPAD // RBLK,),
    in_specs=[pl.BlockSpec((NW, RBLK), lambda j: (0, j))],
    out_specs=pl.BlockSpec(memory_space=pltpu.SMEM),
    out_shape=jax.ShapeDtypeStruct((1, 1), jnp.int32),
)


def kernel(xv, adj_pos, adj_neg):
    xvp = jnp.pad(xv, (0, VPAD - N_VARS)).reshape(32, WORDS)
    packed = _pack_call(xvp).reshape(WORDS)
    partials = _scatter_kernel(packed, adj_pos.reshape(2 * E),
                               adj_neg.reshape(2 * E))
    m = _reduce_call(partials)
    return m[0, 0].astype(jnp.float32)


# speedup vs baseline: 287.2574x; 1.4294x over previous
import functools

import jax
import jax.numpy as jnp
from jax import lax
from jax.experimental import pallas as pl
from jax.experimental.pallas import tpu as pltpu
from jax.experimental.pallas import tpu_sc as plsc

N_VARS = 100000
N_CLAUSES = 100000
E = 3200000

NC = 2
NS = 16
L = 16
NW = NC * NS

WORDS = 4096
LOG2W = WORDS.bit_length() - 1
VPAD = 32 * WORDS
CPAD = 102400
EPW = E // NW
CHUNK = 2000
NCHUNK = EPW // CHUNK
NPAIR = NCHUNK // 2
UNROLL = 5
VECS = CHUNK // (L * UNROLL)
ZUNROLL = 8
THRESH = 0.50001
RBLK = CPAD // 8


def _pack_body(x_ref, out_ref):
    x = x_ref[...]
    b = jnp.where(x >= THRESH, 1, 0).astype(jnp.int32)
    shifts = lax.broadcasted_iota(jnp.int32, (32, 1), 0)
    out_ref[...] = jnp.sum(b << shifts, axis=0, keepdims=True)


_pack_call = pl.pallas_call(
    _pack_body,
    out_shape=jax.ShapeDtypeStruct((1, WORDS), jnp.int32),
)


_sc_mesh = plsc.VectorSubcoreMesh(core_axis_name="c", subcore_axis_name="s")


@functools.partial(
    pl.kernel,
    out_type=jax.ShapeDtypeStruct((NW, CPAD), jnp.int32),
    mesh=_sc_mesh,
    compiler_params=pltpu.CompilerParams(needs_layout_passes=False),
    scratch_types=[
        pltpu.VMEM((WORDS,), jnp.int32),
        pltpu.VMEM((CPAD,), jnp.int32),
        pltpu.VMEM((CHUNK,), jnp.int32),
        pltpu.VMEM((CHUNK,), jnp.int32),
        pltpu.VMEM((CHUNK,), jnp.int32),
        pltpu.VMEM((CHUNK,), jnp.int32),
        pltpu.SemaphoreType.DMA,
        pltpu.SemaphoreType.DMA,
        pltpu.SemaphoreType.DMA,
        pltpu.SemaphoreType.DMA,
    ],
)
def _scatter_kernel(packed_hbm, pos_hbm, neg_hbm, out_hbm,
                    packed_v, acc_v, ic0_v, iv0_v, ic1_v, iv1_v,
                    sc0, sv0, sc1, sv1):
    wid = lax.axis_index("s") * NC + lax.axis_index("c")
    base = wid * EPW
    ic_b, iv_b = (ic0_v, ic1_v), (iv0_v, iv1_v)
    sc_b, sv_b = (sc0, sc1), (sv0, sv1)

    def start(adj_hbm, bi, k):
        off = base + k * CHUNK
        pltpu.async_copy(adj_hbm.at[pl.ds(off, CHUNK)], ic_b[bi], sc_b[bi])
        pltpu.async_copy(adj_hbm.at[pl.ds(E + off, CHUNK)], iv_b[bi], sv_b[bi])

    def wait(adj_hbm, bi):
        pltpu.make_async_copy(adj_hbm.at[pl.ds(base, CHUNK)],
                              ic_b[bi], sc_b[bi]).wait()
        pltpu.make_async_copy(adj_hbm.at[pl.ds(base, CHUNK)],
                              iv_b[bi], sv_b[bi]).wait()

    def process(bi, is_pos):
        def vbody(i, _):
            for u in range(UNROLL):
                o = i * (L * UNROLL) + u * L
                iv = iv_b[bi][pl.ds(o, L)]
                ic = ic_b[bi][pl.ds(o, L)]
                word = plsc.load_gather(packed_v, [iv & (WORDS - 1)])
                sh = lax.shift_right_logical(iv, LOG2W)
                b = lax.shift_right_logical(word, sh) & 1
                val = b if is_pos else 1 - b
                plsc.addupdate_scatter(acc_v, [ic], val)
            return 0

        lax.fori_loop(0, VECS, vbody, 0)

    start(pos_hbm, 0, 0)
    pltpu.sync_copy(packed_hbm, packed_v)

    zeros = jnp.zeros((L,), jnp.int32)

    def zbody(i, _):
        for u in range(ZUNROLL):
            acc_v[pl.ds((i * ZUNROLL + u) * L, L)] = zeros
        return 0

    lax.fori_loop(0, CPAD // (L * ZUNROLL), zbody, 0)

    for adj_hbm, is_pos in ((pos_hbm, True), (neg_hbm, False)):
        def pair(p, _, adj_hbm=adj_hbm, is_pos=is_pos):
            wait(adj_hbm, 0)
            start(adj_hbm, 1, 2 * p + 1)
            process(0, is_pos)
            wait(adj_hbm, 1)

            @pl.when(p < NPAIR - 1)
            def _():
                start(adj_hbm, 0, 2 * p + 2)

            if is_pos:
                @pl.when(p == NPAIR - 1)
                def _():
                    start(neg_hbm, 0, 0)

            process(1, is_pos)
            return 0

        lax.fori_loop(0, NPAIR, pair, 0)

    pltpu.sync_copy(acc_v, out_hbm.at[wid])


def _reduce_body(x_ref, out_ref):
    j = pl.program_id(0)
    s = jnp.sum(x_ref[...], axis=0, keepdims=True)
    cid = j * RBLK + lax.broadcasted_iota(jnp.int32, (1, RBLK), 1)
    s = jnp.where(cid < N_CLAUSES, s, jnp.int32(2**31 - 1))
    m = jnp.min(s)

    @pl.when(j == 0)
    def _():
        out_ref[0, 0] = m

    @pl.when(j > 0)
    def _():
        out_ref[0, 0] = jnp.minimum(out_ref[0, 0], m)


_reduce_call = pl.pallas_call(
    _reduce_body,
    grid=(CPAD // RBLK,),
    in_specs=[pl.BlockSpec((NW, RBLK), lambda j: (0, j))],
    out_specs=pl.BlockSpec(memory_space=pltpu.SMEM),
    out_shape=jax.ShapeDtypeStruct((1, 1), jnp.int32),
)


def kernel(xv, adj_pos, adj_neg):
    xvp = jnp.pad(xv, (0, VPAD - N_VARS)).reshape(32, WORDS)
    packed = _pack_call(xvp).reshape(WORDS)
    partials = _scatter_kernel(packed, adj_pos.reshape(2 * E),
                               adj_neg.reshape(2 * E))
    m = _reduce_call(partials)
    return m[0, 0].astype(jnp.float32)
